# Initial kernel scaffold; baseline (speedup 1.0000x reference)
import jax
import jax.numpy as jnp
from jax.experimental import pallas as pl


def kernel(x, edge_index):
    raise NotImplementedError("write your pallas kernel here")


# speedup vs baseline: 3.0947x; 3.0947x over previous
import functools

import jax
import jax.numpy as jnp
from jax import lax
from jax.experimental import pallas as pl
from jax.experimental.pallas import tpu as pltpu
from jax.experimental.pallas import tpu_sc as plsc

N = 10000
E = 320000
D = 128

NC = 2
NS = 16
NW = NC * NS
L = 16

C = 128
NUM_CHUNKS = E // C
MAX_ITERS = (NUM_CHUNKS + NW - 1) // NW


def _body(x_hbm, src_hbm, dst_hbm, out_hbm,
          sidx, didx, srows, drows, outb, sem_s, sem_d):
    wid = lax.axis_index("s") * NC + lax.axis_index("c")
    lanes = lax.iota(jnp.int32, L)

    def chunk_body(i, carry):
        c = wid + i * NW

        @pl.when(c < NUM_CHUNKS)
        def _():
            base = pl.multiple_of(c * C, C)
            pltpu.sync_copy(src_hbm.at[pl.ds(base, C)], sidx)
            pltpu.sync_copy(dst_hbm.at[pl.ds(base, C)], didx)
            cp_s = pltpu.async_copy(x_hbm.at[sidx], srows, sem_s)
            cp_d = pltpu.async_copy(x_hbm.at[didx], drows, sem_d)
            cp_s.wait()
            cp_d.wait()

            def group(g, carry2):
                res = jnp.zeros((L,), jnp.float32)
                for k in range(L):
                    e = g * L + k
                    acc = srows[e, pl.ds(0, L)] * drows[e, pl.ds(0, L)]
                    for j in range(1, D // L):
                        acc = acc + srows[e, pl.ds(j * L, L)] * drows[e, pl.ds(j * L, L)]
                    for sh in (8, 4, 2, 1):
                        acc = acc + acc.at[(lanes + sh) % L].get(
                            mode="promise_in_bounds")
                    res = jnp.where(lanes == k, acc, res)
                score = 1.0 / (1.0 + jnp.exp(-res))
                outb[pl.ds(g * L, L)] = score
                return carry2

            lax.fori_loop(0, C // L, group, 0)
            pltpu.sync_copy(outb, out_hbm.at[pl.ds(base, C)])

        return carry

    lax.fori_loop(0, MAX_ITERS, chunk_body, 0)


@jax.jit
def _sc_score(x, src, dst):
    mesh = plsc.VectorSubcoreMesh(core_axis_name="c", subcore_axis_name="s")
    f = pl.kernel(
        _body,
        out_type=jax.ShapeDtypeStruct((E,), jnp.float32),
        mesh=mesh,
        scratch_types=[
            pltpu.VMEM((C,), jnp.int32),
            pltpu.VMEM((C,), jnp.int32),
            pltpu.VMEM((C, D), jnp.float32),
            pltpu.VMEM((C, D), jnp.float32),
            pltpu.VMEM((C,), jnp.float32),
            pltpu.SemaphoreType.DMA,
            pltpu.SemaphoreType.DMA,
        ],
    )
    return f(x, src, dst)


def kernel(x, edge_index):
    src = edge_index[0]
    dst = edge_index[1]
    return _sc_score(x, src, dst)


# speedup vs baseline: 4.9058x; 1.5852x over previous
import jax
import jax.numpy as jnp
from jax import lax
from jax.experimental import pallas as pl
from jax.experimental.pallas import tpu as pltpu
from jax.experimental.pallas import tpu_sc as plsc

N = 10000
E = 320000
D = 128

NC = 2
NS = 16
NW = NC * NS
L = 16

C = 80
NUM_CHUNKS = E // C
CPW = NUM_CHUNKS // NW
G = C // L


def _body(x_hbm, src_hbm, dst_hbm, out_hbm,
          sidx, didx, srows0, srows1, drows0, drows1, outb0, outb1,
          sem_g0, sem_g1, sem_o0, sem_o1):
    wid = lax.axis_index("s") * NC + lax.axis_index("c")
    lanes = lax.iota(jnp.int32, L)
    rows = (srows0, srows1), (drows0, drows1)
    outs = (outb0, outb1)
    gsems = (sem_g0, sem_g1)
    osems = (sem_o0, sem_o1)

    first = wid * CPW * C
    pltpu.sync_copy(src_hbm.at[pl.ds(first, CPW * C)], sidx)
    pltpu.sync_copy(dst_hbm.at[pl.ds(first, CPW * C)], didx)

    def islice(ref, it):
        return ref.at[pl.ds(pl.multiple_of(it * C, 8), C)]

    def fire(it, b):
        pltpu.async_copy(x_hbm.at[islice(sidx, it)], rows[0][b], gsems[b])
        pltpu.async_copy(x_hbm.at[islice(didx, it)], rows[1][b], gsems[b])

    def wait_gathers(it, b):
        pltpu.make_async_copy(x_hbm.at[islice(sidx, it)], rows[0][b], gsems[b]).wait()
        pltpu.make_async_copy(x_hbm.at[islice(didx, it)], rows[1][b], gsems[b]).wait()

    def out_base(it):
        return pl.multiple_of(first + it * C, C)

    def fire_out(it, b):
        pltpu.async_copy(outs[b], out_hbm.at[pl.ds(out_base(it), C)], osems[b])

    def wait_out(it, b):
        pltpu.make_async_copy(
            outs[b], out_hbm.at[pl.ds(out_base(it), C)], osems[b]).wait()

    def compute(it, b):
        srows, drows, outb = rows[0][b], rows[1][b], outs[b]

        def group(g, carry):
            res = jnp.zeros((L,), jnp.float32)
            for k in range(L):
                e = g * L + k
                acc = srows[e, pl.ds(0, L)] * drows[e, pl.ds(0, L)]
                for j in range(1, D // L):
                    acc = acc + srows[e, pl.ds(j * L, L)] * drows[e, pl.ds(j * L, L)]
                for sh in (8, 4, 2, 1):
                    acc = acc + acc.at[(lanes + sh) % L].get(
                        mode="promise_in_bounds")
                res = jnp.where(lanes == k, acc, res)
            outb[pl.ds(g * L, L)] = 1.0 / (1.0 + jnp.exp(-res))
            return carry

        lax.fori_loop(0, G, group, 0)

    fire(0, 0)

    def pair(j, carry):
        it = 2 * j

        @pl.when(j < (CPW - 1) // 2)
        def _():
            fire(it + 1, 1)

        wait_gathers(it, 0)

        @pl.when(j >= 1)
        def _():
            wait_out(it - 2, 0)

        compute(it, 0)
        fire_out(it, 0)

        @pl.when(j < (CPW - 1) // 2)
        def _():
            it1 = 2 * j + 1
            fire(it1 + 1, 0)
            wait_gathers(it1, 1)

            @pl.when(j >= 1)
            def _():
                wait_out(it1 - 2, 1)

            compute(it1, 1)
            fire_out(it1, 1)

        return carry

    lax.fori_loop(0, (CPW + 1) // 2, pair, 0)

    wait_out(CPW - 1, 0)
    wait_out(CPW - 2, 1)


@jax.jit
def _sc_score(x, src, dst):
    mesh = plsc.VectorSubcoreMesh(core_axis_name="c", subcore_axis_name="s")
    f = pl.kernel(
        _body,
        out_type=jax.ShapeDtypeStruct((E,), jnp.float32),
        mesh=mesh,
        scratch_types=[
            pltpu.VMEM((CPW * C,), jnp.int32),
            pltpu.VMEM((CPW * C,), jnp.int32),
            pltpu.VMEM((C, D), jnp.float32),
            pltpu.VMEM((C, D), jnp.float32),
            pltpu.VMEM((C, D), jnp.float32),
            pltpu.VMEM((C, D), jnp.float32),
            pltpu.VMEM((C,), jnp.float32),
            pltpu.VMEM((C,), jnp.float32),
            pltpu.SemaphoreType.DMA,
            pltpu.SemaphoreType.DMA,
            pltpu.SemaphoreType.DMA,
            pltpu.SemaphoreType.DMA,
        ],
    )
    return f(x, src, dst)


def kernel(x, edge_index):
    src = edge_index[0]
    dst = edge_index[1]
    return _sc_score(x, src, dst)
